# baseline (speedup 1.0000x reference)
import jax
import jax.numpy as jnp
from jax import lax
from jax.experimental import pallas as pl
from jax.experimental.pallas import tpu as pltpu
from jax.experimental.pallas import tpu_sc as plsc

N = 10000
NP = 10240
E = 320000
D = 128
G = 100

NC = 2
NS = 16
NW = NC * NS
EPT = E // NW
K = 80
NCHUNK = EPT // K
BPT = NP // NW
NBCH = BPT // K
ROWS_PT = NP // NS
CNT_PAD = 256
SP_PAD = G * NP
SP_SLICE = SP_PAD // NS
RB = NP // 8
NBLK = NP // RB


def _sc_mesh():
  return plsc.VectorSubcoreMesh(
      core_axis_name="c", subcore_axis_name="s", num_cores=NC, num_subcores=NS)


def _sc_params():
  return pltpu.CompilerParams(needs_layout_passes=False)


def _sc_hist_body(dst3, batch3, z640, z256, deg_out, cnt_out,
                  deg_sh, cnt_sh, dst_v, b_v, ones_v):
  c = lax.axis_index("c")
  s = lax.axis_index("s")
  w = c * NS + s

  pltpu.sync_copy(z640, deg_sh.at[pl.ds(s * 640, 640)])

  @pl.when(s == 0)
  def _():
    pltpu.sync_copy(z256, cnt_sh)

  pltpu.sync_copy(dst3.at[w], dst_v)
  pltpu.sync_copy(batch3.at[w], b_v)
  for k in range(K // 16):
    ones_v[pl.ds(k * 16, 16)] = jnp.ones((16,), jnp.float32)

  plsc.subcore_barrier()

  @pl.loop(0, NCHUNK)
  def _(j):
    pltpu.sync_copy(ones_v, deg_sh.at[dst_v.at[j]], add=True)

  @pl.loop(0, NBCH)
  def _(j):
    pltpu.sync_copy(ones_v, cnt_sh.at[b_v.at[j]], add=True)

  plsc.subcore_barrier()

  pltpu.sync_copy(deg_sh.at[pl.ds(s * 640, 640)],
                  deg_out.at[c, pl.ds(s * 640, 640)])

  @pl.when(s == 0)
  def _():
    pltpu.sync_copy(cnt_sh, cnt_out.at[c])


def _sc_hist(dst3, batch3, z640, z256):
  return pl.kernel(
      _sc_hist_body,
      out_type=[
          jax.ShapeDtypeStruct((NC, NP), jnp.float32),
          jax.ShapeDtypeStruct((NC, CNT_PAD), jnp.float32),
      ],
      mesh=_sc_mesh(),
      scratch_types=[
          pltpu.VMEM_SHARED((NP,), jnp.float32),
          pltpu.VMEM_SHARED((CNT_PAD,), jnp.float32),
          pltpu.VMEM((NCHUNK, K), jnp.int32),
          pltpu.VMEM((NBCH, K), jnp.int32),
          pltpu.VMEM((K,), jnp.float32),
      ],
      compiler_params=_sc_params(),
      name="sc_hist",
  )(dst3, batch3, z640, z256)


NCHB = 128
PH = 64


def _db_run(u, acc_sh, src_v, dst_v, rows_a, rows_b, sem_a, sem_b, count):
  pltpu.async_copy(u.at[src_v.at[0]], rows_a, sem_a)
  npairs = (count - 2) // 2

  @pl.loop(0, npairs)
  def _(i):
    j0 = 2 * i
    pltpu.async_copy(u.at[src_v.at[j0 + 1]], rows_b, sem_b)
    pltpu.make_async_copy(u.at[src_v.at[j0]], rows_a, sem_a).wait()
    pltpu.sync_copy(rows_a, acc_sh.at[dst_v.at[j0]], add=True)
    pltpu.async_copy(u.at[src_v.at[j0 + 2]], rows_a, sem_a)
    pltpu.make_async_copy(u.at[src_v.at[j0 + 1]], rows_b, sem_b).wait()
    pltpu.sync_copy(rows_b, acc_sh.at[dst_v.at[j0 + 1]], add=True)

  if count % 2 == 0:
    pltpu.async_copy(u.at[src_v.at[count - 1]], rows_b, sem_b)
    pltpu.make_async_copy(u.at[src_v.at[count - 2]], rows_a, sem_a).wait()
    pltpu.sync_copy(rows_a, acc_sh.at[dst_v.at[count - 2]], add=True)
    pltpu.make_async_copy(u.at[src_v.at[count - 1]], rows_b, sem_b).wait()
    pltpu.sync_copy(rows_b, acc_sh.at[dst_v.at[count - 1]], add=True)
  else:
    pltpu.async_copy(u.at[src_v.at[count - 2]], rows_b, sem_b)
    pltpu.make_async_copy(u.at[src_v.at[count - 3]], rows_a, sem_a).wait()
    pltpu.sync_copy(rows_a, acc_sh.at[dst_v.at[count - 3]], add=True)
    pltpu.async_copy(u.at[src_v.at[count - 1]], rows_a, sem_a)
    pltpu.make_async_copy(u.at[src_v.at[count - 2]], rows_b, sem_b).wait()
    pltpu.sync_copy(rows_b, acc_sh.at[dst_v.at[count - 2]], add=True)
    pltpu.make_async_copy(u.at[src_v.at[count - 1]], rows_a, sem_a).wait()
    pltpu.sync_copy(rows_a, acc_sh.at[dst_v.at[count - 1]], add=True)


def _sc_edge_agg_body(src3, dst3, u, acc_out,
                      acc_sh, src_v, dst_v, rows_a, rows_b, sem_a, sem_b):
  c = lax.axis_index("c")
  s = lax.axis_index("s")
  w = c * NS + s

  @pl.loop(0, K)
  def _(q):
    for l in range(D // 16):
      rows_a[q, pl.ds(l * 16, 16)] = jnp.zeros((16,), jnp.float32)

  for q in range(ROWS_PT // K):
    pltpu.sync_copy(rows_a, acc_sh.at[pl.ds(s * ROWS_PT + q * K, K)])

  pltpu.sync_copy(src3.at[w, pl.ds(0, PH)], src_v)
  pltpu.sync_copy(dst3.at[w, pl.ds(0, PH)], dst_v)

  plsc.subcore_barrier()

  _db_run(u, acc_sh, src_v, dst_v, rows_a, rows_b, sem_a, sem_b, PH)

  pltpu.sync_copy(src3.at[w, pl.ds(PH, PH)], src_v)
  pltpu.sync_copy(dst3.at[w, pl.ds(PH, PH)], dst_v)
  _db_run(u, acc_sh, src_v, dst_v, rows_a, rows_b, sem_a, sem_b, PH)

  plsc.subcore_barrier()

  for q in range(ROWS_PT // K):
    pltpu.sync_copy(acc_sh.at[pl.ds(s * ROWS_PT + q * K, K)], rows_a)
    pltpu.sync_copy(rows_a, acc_out.at[c, pl.ds(s * ROWS_PT + q * K, K)])


def _sc_edge_agg(src3, dst3, u):
  return pl.kernel(
      _sc_edge_agg_body,
      out_type=jax.ShapeDtypeStruct((NC, NP, D), jnp.float32),
      mesh=_sc_mesh(),
      scratch_types=[
          pltpu.VMEM_SHARED((NP, D), jnp.float32),
          pltpu.VMEM((PH, K), jnp.int32),
          pltpu.VMEM((PH, K), jnp.int32),
          pltpu.VMEM((K, D), jnp.float32),
          pltpu.VMEM((K, D), jnp.float32),
          pltpu.SemaphoreType.DMA,
          pltpu.SemaphoreType.DMA,
      ],
      compiler_params=_sc_params(),
      name="sc_edge_agg",
  )(src3, dst3, u)


def _sc_sp_body(src3, dst3, batch_h, dinv_h, cntinv_h, zsp, sp_out,
                sp_sh, src_v, dst_v, batch_v, dinv_v, cntinv_v, fbuf, vbuf):
  c = lax.axis_index("c")
  s = lax.axis_index("s")
  w = c * NS + s

  pltpu.sync_copy(zsp, sp_sh.at[pl.ds(s * SP_SLICE, SP_SLICE)])
  pltpu.sync_copy(src3.at[w], src_v)
  pltpu.sync_copy(dst3.at[w], dst_v)
  pltpu.sync_copy(batch_h, batch_v)
  pltpu.sync_copy(dinv_h, dinv_v)
  pltpu.sync_copy(cntinv_h, cntinv_v)

  plsc.subcore_barrier()

  @pl.loop(0, NCHUNK)
  def _(j):
    for k in range(K // 16):
      sv = src_v[j, pl.ds(k * 16, 16)]
      dv = dst_v[j, pl.ds(k * 16, 16)]
      a_s = plsc.load_gather(dinv_v, [sv])
      a_d = plsc.load_gather(dinv_v, [dv])
      g = plsc.load_gather(batch_v, [dv])
      ci = plsc.load_gather(cntinv_v, [g])
      vbuf[pl.ds(k * 16, 16)] = a_s * a_d * ci
      fbuf[pl.ds(k * 16, 16)] = g * NP + sv
    pltpu.sync_copy(vbuf, sp_sh.at[fbuf], add=True)

  plsc.subcore_barrier()

  pltpu.sync_copy(sp_sh.at[pl.ds(s * SP_SLICE, SP_SLICE)],
                  sp_out.at[c, pl.ds(s * SP_SLICE, SP_SLICE)])


def _sc_sp(src3, dst3, batch_h, dinv_h, cntinv_h, zsp):
  return pl.kernel(
      _sc_sp_body,
      out_type=jax.ShapeDtypeStruct((NC, SP_PAD), jnp.float32),
      mesh=_sc_mesh(),
      scratch_types=[
          pltpu.VMEM_SHARED((SP_PAD,), jnp.float32),
          pltpu.VMEM((NCHUNK, K), jnp.int32),
          pltpu.VMEM((NCHUNK, K), jnp.int32),
          pltpu.VMEM((NP,), jnp.int32),
          pltpu.VMEM((NP,), jnp.float32),
          pltpu.VMEM((CNT_PAD,), jnp.float32),
          pltpu.VMEM((K,), jnp.int32),
          pltpu.VMEM((K,), jnp.float32),
      ],
      compiler_params=_sc_params(),
      name="sc_sp",
  )(src3, dst3, batch_h, dinv_h, cntinv_h, zsp)


def _tc_prep_body(x_ref, w1_ref, deg_ref, cnt_ref, batch_ref,
                  u_ref, dinv_ref, ac_ref, cntinv_ref):
  i = pl.program_id(0)
  deg = deg_ref[0, 0, 0, :] + deg_ref[1, 0, 0, :] + 1.0
  dinv = lax.rsqrt(deg)
  xw = jnp.dot(x_ref[...], w1_ref[...],
               preferred_element_type=jnp.float32)
  u_ref[...] = dinv[:, None] * xw
  dinv_ref[0, 0, :] = dinv

  cntsum = cnt_ref[0, :] + cnt_ref[1, :]
  cntinv = 1.0 / jnp.maximum(cntsum, 1.0)

  @pl.when(i == 0)
  def _():
    cntinv_ref[0, :] = cntinv

  oh = (batch_ref[0, 0, :][:, None]
        == lax.broadcasted_iota(jnp.int32, (RB, CNT_PAD), 1))
  cpn = jnp.dot(oh.astype(jnp.float32), cntinv[:, None],
                preferred_element_type=jnp.float32)[:, 0]
  ac_ref[0, 0, :] = dinv * dinv * cpn


def _tc_prep(xp, W1, deg4, cnt, batch3d):
  return pl.pallas_call(
      _tc_prep_body,
      grid=(NBLK,),
      in_specs=[
          pl.BlockSpec((RB, D), lambda i: (i, 0)),
          pl.BlockSpec((D, D), lambda i: (0, 0)),
          pl.BlockSpec((NC, 1, 1, RB), lambda i: (0, i, 0, 0)),
          pl.BlockSpec((NC, CNT_PAD), lambda i: (0, 0)),
          pl.BlockSpec((1, 1, RB), lambda i: (i, 0, 0)),
      ],
      out_specs=[
          pl.BlockSpec((RB, D), lambda i: (i, 0)),
          pl.BlockSpec((1, 1, RB), lambda i: (i, 0, 0)),
          pl.BlockSpec((1, 1, RB), lambda i: (i, 0, 0)),
          pl.BlockSpec((1, CNT_PAD), lambda i: (0, 0)),
      ],
      out_shape=[
          jax.ShapeDtypeStruct((NP, D), jnp.float32),
          jax.ShapeDtypeStruct((NBLK, 1, RB), jnp.float32),
          jax.ShapeDtypeStruct((NBLK, 1, RB), jnp.float32),
          jax.ShapeDtypeStruct((1, CNT_PAD), jnp.float32),
      ],
      name="tc_prep",
  )(xp, W1, deg4, cnt, batch3d)


def _tc_pool_body(acc_ref, u_ref, dinv_ref, ac_ref, batch_ref, sp_ref,
                  cnt_ref, b1_ref, w2_ref, b2_ref, out_ref, accum):
  i = pl.program_id(0)
  accs = acc_ref[0] + acc_ref[1] + u_ref[...]
  h1 = jnp.maximum(dinv_ref[0, 0, :][:, None] * accs
                   + b1_ref[0, :][None, :], 0.0)
  iota_g = lax.broadcasted_iota(jnp.int32, (G, RB), 0)
  spb = sp_ref[0] + sp_ref[1] + jnp.where(
      batch_ref[0, 0, :][None, :] == iota_g,
      ac_ref[0, 0, :][None, :], 0.0)
  part = jnp.dot(spb, h1, preferred_element_type=jnp.float32)

  @pl.when(i == 0)
  def _():
    accum[...] = jnp.zeros_like(accum)

  accum[...] += part

  @pl.when(i == pl.num_programs(0) - 1)
  def _():
    cntsum = cnt_ref[0, :G] + cnt_ref[1, :G]
    mask = (cntsum > 0.0).astype(jnp.float32)[:, None]
    out_ref[...] = (jnp.dot(accum[...], w2_ref[...],
                            preferred_element_type=jnp.float32)
                    + mask * b2_ref[0, :][None, :])


def _tc_pool(acc_pb, u, dinv3, ac3, batch3d, sp3, cnt, b1r, W2, b2r):
  return pl.pallas_call(
      _tc_pool_body,
      grid=(NBLK,),
      in_specs=[
          pl.BlockSpec((NC, RB, D), lambda i: (0, i, 0)),
          pl.BlockSpec((RB, D), lambda i: (i, 0)),
          pl.BlockSpec((1, 1, RB), lambda i: (i, 0, 0)),
          pl.BlockSpec((1, 1, RB), lambda i: (i, 0, 0)),
          pl.BlockSpec((1, 1, RB), lambda i: (i, 0, 0)),
          pl.BlockSpec((NC, G, RB), lambda i: (0, 0, i)),
          pl.BlockSpec((NC, CNT_PAD), lambda i: (0, 0)),
          pl.BlockSpec((1, D), lambda i: (0, 0)),
          pl.BlockSpec((D, D), lambda i: (0, 0)),
          pl.BlockSpec((1, D), lambda i: (0, 0)),
      ],
      out_specs=pl.BlockSpec((G, D), lambda i: (0, 0)),
      out_shape=jax.ShapeDtypeStruct((G, D), jnp.float32),
      scratch_shapes=[pltpu.VMEM((G, D), jnp.float32)],
      name="tc_pool",
  )(acc_pb, u, dinv3, ac3, batch3d, sp3, cnt, b1r, W2, b2r)


def kernel(x, edge_index, batch, W1, b1, W2, b2):
  src = edge_index[0]
  dst = edge_index[1]
  src3 = src.reshape(NW, NCHUNK, K)
  dst3 = dst.reshape(NW, NCHUNK, K)
  npad = NW * NCHB * K - E
  srcb = jnp.concatenate([src, jnp.full((npad,), N, jnp.int32)])
  dstb = jnp.concatenate([dst, jnp.full((npad,), N, jnp.int32)])
  src3b = srcb.reshape(NW, NCHB, K)
  dst3b = dstb.reshape(NW, NCHB, K)
  batch_p = jnp.concatenate([batch, jnp.full((NP - N,), G, jnp.int32)])
  batch3 = batch_p.reshape(NW, NBCH, K)
  batch3d = batch_p.reshape(NBLK, 1, RB)
  xp = jnp.concatenate([x, jnp.zeros((NP - N, D), jnp.float32)], axis=0)

  z640 = jnp.zeros((640,), jnp.float32)
  z256 = jnp.zeros((CNT_PAD,), jnp.float32)
  zsp = jnp.zeros((SP_SLICE,), jnp.float32)

  deg_pb, cnt_pb = _sc_hist(dst3, batch3, z640, z256)
  deg4 = deg_pb.reshape(NC, NBLK, 1, RB)

  u, dinv3, ac3, cntinv = _tc_prep(xp, W1, deg4, cnt_pb, batch3d)

  acc_pb = _sc_edge_agg(src3b, dst3b, u)
  sp_pb = _sc_sp(src3, dst3, batch_p, dinv3.reshape(NP),
                 cntinv.reshape(CNT_PAD), zsp)
  sp3 = sp_pb.reshape(NC, G, NP)

  return _tc_pool(acc_pb, u, dinv3, ac3, batch3d, sp3, cnt_pb,
                  b1.reshape(1, D), W2, b2.reshape(1, D))


# speedup vs baseline: 2.1339x; 2.1339x over previous
import jax
import jax.numpy as jnp
from jax import lax
from jax.experimental import pallas as pl
from jax.experimental.pallas import tpu as pltpu
from jax.experimental.pallas import tpu_sc as plsc

N = 10000
NP = 10240
E = 320000
D = 128
G = 100

NC = 2
NS = 16
NW = NC * NS
EPT = E // NW
K = 80
NCHUNK = EPT // K
BPT = NP // NW
NBCH = BPT // K
ROWS_PT = NP // NS
CNT_PAD = 256
SP_PAD = G * NP
SP_SLICE = SP_PAD // NS
RB = NP // 8
NBLK = NP // RB


def _sc_mesh():
  return plsc.VectorSubcoreMesh(
      core_axis_name="c", subcore_axis_name="s", num_cores=NC, num_subcores=NS)


def _sc_params():
  return pltpu.CompilerParams(needs_layout_passes=False)


def _sc_hist_body(dst3, batch3, z640, z256, deg_out, cnt_out,
                  deg_sh, cnt_sh, dst_v, b_v, ones_v):
  c = lax.axis_index("c")
  s = lax.axis_index("s")
  w = c * NS + s

  pltpu.sync_copy(z640, deg_sh.at[pl.ds(s * 640, 640)])

  @pl.when(s == 0)
  def _():
    pltpu.sync_copy(z256, cnt_sh)

  pltpu.sync_copy(dst3.at[w], dst_v)
  pltpu.sync_copy(batch3.at[w], b_v)
  for k in range(K // 16):
    ones_v[pl.ds(k * 16, 16)] = jnp.ones((16,), jnp.float32)

  plsc.subcore_barrier()

  @pl.loop(0, NCHUNK)
  def _(j):
    pltpu.sync_copy(ones_v, deg_sh.at[dst_v.at[j]], add=True)

  @pl.loop(0, NBCH)
  def _(j):
    pltpu.sync_copy(ones_v, cnt_sh.at[b_v.at[j]], add=True)

  plsc.subcore_barrier()

  pltpu.sync_copy(deg_sh.at[pl.ds(s * 640, 640)],
                  deg_out.at[c, pl.ds(s * 640, 640)])

  @pl.when(s == 0)
  def _():
    pltpu.sync_copy(cnt_sh, cnt_out.at[c])


def _sc_hist(dst3, batch3, z640, z256):
  return pl.kernel(
      _sc_hist_body,
      out_type=[
          jax.ShapeDtypeStruct((NC, NP), jnp.float32),
          jax.ShapeDtypeStruct((NC, CNT_PAD), jnp.float32),
      ],
      mesh=_sc_mesh(),
      scratch_types=[
          pltpu.VMEM_SHARED((NP,), jnp.float32),
          pltpu.VMEM_SHARED((CNT_PAD,), jnp.float32),
          pltpu.VMEM((NCHUNK, K), jnp.int32),
          pltpu.VMEM((NBCH, K), jnp.int32),
          pltpu.VMEM((K,), jnp.float32),
      ],
      compiler_params=_sc_params(),
      name="sc_hist",
  )(dst3, batch3, z640, z256)


NCHB = 128
PH = 64


def _db_run(u, acc_sh, src_v, dst_v, rows_a, rows_b, sem_a, sem_b, count):
  pltpu.async_copy(u.at[src_v.at[0]], rows_a, sem_a)
  npairs = (count - 2) // 2

  @pl.loop(0, npairs)
  def _(i):
    j0 = 2 * i
    pltpu.async_copy(u.at[src_v.at[j0 + 1]], rows_b, sem_b)
    pltpu.make_async_copy(u.at[src_v.at[j0]], rows_a, sem_a).wait()
    pltpu.sync_copy(rows_a, acc_sh.at[dst_v.at[j0]], add=True)
    pltpu.async_copy(u.at[src_v.at[j0 + 2]], rows_a, sem_a)
    pltpu.make_async_copy(u.at[src_v.at[j0 + 1]], rows_b, sem_b).wait()
    pltpu.sync_copy(rows_b, acc_sh.at[dst_v.at[j0 + 1]], add=True)

  if count % 2 == 0:
    pltpu.async_copy(u.at[src_v.at[count - 1]], rows_b, sem_b)
    pltpu.make_async_copy(u.at[src_v.at[count - 2]], rows_a, sem_a).wait()
    pltpu.sync_copy(rows_a, acc_sh.at[dst_v.at[count - 2]], add=True)
    pltpu.make_async_copy(u.at[src_v.at[count - 1]], rows_b, sem_b).wait()
    pltpu.sync_copy(rows_b, acc_sh.at[dst_v.at[count - 1]], add=True)
  else:
    pltpu.async_copy(u.at[src_v.at[count - 2]], rows_b, sem_b)
    pltpu.make_async_copy(u.at[src_v.at[count - 3]], rows_a, sem_a).wait()
    pltpu.sync_copy(rows_a, acc_sh.at[dst_v.at[count - 3]], add=True)
    pltpu.async_copy(u.at[src_v.at[count - 1]], rows_a, sem_a)
    pltpu.make_async_copy(u.at[src_v.at[count - 2]], rows_b, sem_b).wait()
    pltpu.sync_copy(rows_b, acc_sh.at[dst_v.at[count - 2]], add=True)
    pltpu.make_async_copy(u.at[src_v.at[count - 1]], rows_a, sem_a).wait()
    pltpu.sync_copy(rows_a, acc_sh.at[dst_v.at[count - 1]], add=True)


def _sc_edge_agg_body(src3, dst3, u, acc_out,
                      acc_sh, src_v, dst_v, rows_a, rows_b, sem_a, sem_b):
  c = lax.axis_index("c")
  s = lax.axis_index("s")
  w = c * NS + s

  @pl.loop(0, K)
  def _(q):
    for l in range(D // 16):
      rows_a[q, pl.ds(l * 16, 16)] = jnp.zeros((16,), jnp.float32)

  for q in range(ROWS_PT // K):
    pltpu.sync_copy(rows_a, acc_sh.at[pl.ds(s * ROWS_PT + q * K, K)])

  pltpu.sync_copy(src3.at[w, pl.ds(0, PH)], src_v)
  pltpu.sync_copy(dst3.at[w, pl.ds(0, PH)], dst_v)

  plsc.subcore_barrier()

  _db_run(u, acc_sh, src_v, dst_v, rows_a, rows_b, sem_a, sem_b, PH)

  pltpu.sync_copy(src3.at[w, pl.ds(PH, PH)], src_v)
  pltpu.sync_copy(dst3.at[w, pl.ds(PH, PH)], dst_v)
  _db_run(u, acc_sh, src_v, dst_v, rows_a, rows_b, sem_a, sem_b, PH)

  plsc.subcore_barrier()

  for q in range(ROWS_PT // K):
    pltpu.sync_copy(acc_sh.at[pl.ds(s * ROWS_PT + q * K, K)], rows_a)
    pltpu.sync_copy(rows_a, acc_out.at[c, pl.ds(s * ROWS_PT + q * K, K)])


def _sc_edge_agg(src3, dst3, u):
  return pl.kernel(
      _sc_edge_agg_body,
      out_type=jax.ShapeDtypeStruct((NC, NP, D), jnp.float32),
      mesh=_sc_mesh(),
      scratch_types=[
          pltpu.VMEM_SHARED((NP, D), jnp.float32),
          pltpu.VMEM((PH, K), jnp.int32),
          pltpu.VMEM((PH, K), jnp.int32),
          pltpu.VMEM((K, D), jnp.float32),
          pltpu.VMEM((K, D), jnp.float32),
          pltpu.SemaphoreType.DMA,
          pltpu.SemaphoreType.DMA,
      ],
      compiler_params=_sc_params(),
      name="sc_edge_agg",
  )(src3, dst3, u)


def _sc_sp_body(src3, dst3, batch_h, dinv_h, cntinv_h, zsp, sp_out,
                sp_sh, src_v, dst_v, batch_v, dinv_v, cntinv_v, fbuf, vbuf):
  c = lax.axis_index("c")
  s = lax.axis_index("s")
  w = c * NS + s

  pltpu.sync_copy(zsp, sp_sh.at[pl.ds(s * SP_SLICE, SP_SLICE)])
  pltpu.sync_copy(src3.at[w], src_v)
  pltpu.sync_copy(dst3.at[w], dst_v)
  pltpu.sync_copy(batch_h, batch_v)
  pltpu.sync_copy(dinv_h, dinv_v)
  pltpu.sync_copy(cntinv_h, cntinv_v)

  plsc.subcore_barrier()

  @pl.loop(0, NCHUNK)
  def _(j):
    for k in range(K // 16):
      sv = src_v[j, pl.ds(k * 16, 16)]
      dv = dst_v[j, pl.ds(k * 16, 16)]
      a_s = plsc.load_gather(dinv_v, [sv])
      a_d = plsc.load_gather(dinv_v, [dv])
      g = plsc.load_gather(batch_v, [dv])
      ci = plsc.load_gather(cntinv_v, [g])
      vbuf[pl.ds(k * 16, 16)] = a_s * a_d * ci
      fbuf[pl.ds(k * 16, 16)] = g * NP + sv
    pltpu.sync_copy(vbuf, sp_sh.at[fbuf], add=True)

  plsc.subcore_barrier()

  pltpu.sync_copy(sp_sh.at[pl.ds(s * SP_SLICE, SP_SLICE)],
                  sp_out.at[c, pl.ds(s * SP_SLICE, SP_SLICE)])


def _sc_sp(src3, dst3, batch_h, dinv_h, cntinv_h, zsp):
  return pl.kernel(
      _sc_sp_body,
      out_type=jax.ShapeDtypeStruct((NC, SP_PAD), jnp.float32),
      mesh=_sc_mesh(),
      scratch_types=[
          pltpu.VMEM_SHARED((SP_PAD,), jnp.float32),
          pltpu.VMEM((NCHUNK, K), jnp.int32),
          pltpu.VMEM((NCHUNK, K), jnp.int32),
          pltpu.VMEM((NP,), jnp.int32),
          pltpu.VMEM((NP,), jnp.float32),
          pltpu.VMEM((CNT_PAD,), jnp.float32),
          pltpu.VMEM((K,), jnp.int32),
          pltpu.VMEM((K,), jnp.float32),
      ],
      compiler_params=_sc_params(),
      name="sc_sp",
  )(src3, dst3, batch_h, dinv_h, cntinv_h, zsp)


def _tc_prep_body(x_ref, w1_ref, deg_ref, cnt_ref, batch_ref,
                  u_ref, dinv_ref, ac_ref, cntinv_ref):
  i = pl.program_id(0)
  deg = deg_ref[0, 0, 0, :] + deg_ref[1, 0, 0, :] + 1.0
  dinv = lax.rsqrt(deg)
  xw = jnp.dot(x_ref[...], w1_ref[...],
               preferred_element_type=jnp.float32)
  u_ref[...] = dinv[:, None] * xw
  dinv_ref[0, 0, :] = dinv

  cntsum = cnt_ref[0, :] + cnt_ref[1, :]
  cntinv = 1.0 / jnp.maximum(cntsum, 1.0)

  @pl.when(i == 0)
  def _():
    cntinv_ref[0, :] = cntinv

  oh = (batch_ref[0, 0, :][:, None]
        == lax.broadcasted_iota(jnp.int32, (RB, CNT_PAD), 1))
  cpn = jnp.dot(oh.astype(jnp.float32), cntinv[:, None],
                preferred_element_type=jnp.float32)[:, 0]
  ac_ref[0, 0, :] = dinv * dinv * cpn


def _tc_prep(xp, W1, deg4, cnt, batch3d):
  return pl.pallas_call(
      _tc_prep_body,
      grid=(NBLK,),
      in_specs=[
          pl.BlockSpec((RB, D), lambda i: (i, 0)),
          pl.BlockSpec((D, D), lambda i: (0, 0)),
          pl.BlockSpec((NC, 1, 1, RB), lambda i: (0, i, 0, 0)),
          pl.BlockSpec((NC, CNT_PAD), lambda i: (0, 0)),
          pl.BlockSpec((1, 1, RB), lambda i: (i, 0, 0)),
      ],
      out_specs=[
          pl.BlockSpec((RB, D), lambda i: (i, 0)),
          pl.BlockSpec((1, 1, RB), lambda i: (i, 0, 0)),
          pl.BlockSpec((1, 1, RB), lambda i: (i, 0, 0)),
          pl.BlockSpec((1, CNT_PAD), lambda i: (0, 0)),
      ],
      out_shape=[
          jax.ShapeDtypeStruct((NP, D), jnp.float32),
          jax.ShapeDtypeStruct((NBLK, 1, RB), jnp.float32),
          jax.ShapeDtypeStruct((NBLK, 1, RB), jnp.float32),
          jax.ShapeDtypeStruct((1, CNT_PAD), jnp.float32),
      ],
      name="tc_prep",
  )(xp, W1, deg4, cnt, batch3d)


def _tc_pool_body(acc_ref, u_ref, dinv_ref, ac_ref, batch_ref, sp_ref,
                  cnt_ref, b1_ref, w2_ref, b2_ref, out_ref, accum):
  i = pl.program_id(0)
  accs = acc_ref[0] + acc_ref[1] + u_ref[...]
  h1 = jnp.maximum(dinv_ref[0, 0, :][:, None] * accs
                   + b1_ref[0, :][None, :], 0.0)
  iota_g = lax.broadcasted_iota(jnp.int32, (G, RB), 0)
  spb = sp_ref[0] + sp_ref[1] + jnp.where(
      batch_ref[0, 0, :][None, :] == iota_g,
      ac_ref[0, 0, :][None, :], 0.0)
  part = jnp.dot(spb, h1, preferred_element_type=jnp.float32)

  @pl.when(i == 0)
  def _():
    accum[...] = jnp.zeros_like(accum)

  accum[...] += part

  @pl.when(i == pl.num_programs(0) - 1)
  def _():
    cntsum = cnt_ref[0, :G] + cnt_ref[1, :G]
    mask = (cntsum > 0.0).astype(jnp.float32)[:, None]
    out_ref[...] = (jnp.dot(accum[...], w2_ref[...],
                            preferred_element_type=jnp.float32)
                    + mask * b2_ref[0, :][None, :])


def _tc_pool(acc_pb, u, dinv3, ac3, batch3d, sp3, cnt, b1r, W2, b2r):
  return pl.pallas_call(
      _tc_pool_body,
      grid=(NBLK,),
      in_specs=[
          pl.BlockSpec((NC, RB, D), lambda i: (0, i, 0)),
          pl.BlockSpec((RB, D), lambda i: (i, 0)),
          pl.BlockSpec((1, 1, RB), lambda i: (i, 0, 0)),
          pl.BlockSpec((1, 1, RB), lambda i: (i, 0, 0)),
          pl.BlockSpec((1, 1, RB), lambda i: (i, 0, 0)),
          pl.BlockSpec((NC, G, RB), lambda i: (0, 0, i)),
          pl.BlockSpec((NC, CNT_PAD), lambda i: (0, 0)),
          pl.BlockSpec((1, D), lambda i: (0, 0)),
          pl.BlockSpec((D, D), lambda i: (0, 0)),
          pl.BlockSpec((1, D), lambda i: (0, 0)),
      ],
      out_specs=pl.BlockSpec((G, D), lambda i: (0, 0)),
      out_shape=jax.ShapeDtypeStruct((G, D), jnp.float32),
      scratch_shapes=[pltpu.VMEM((G, D), jnp.float32)],
      name="tc_pool",
  )(acc_pb, u, dinv3, ac3, batch3d, sp3, cnt, b1r, W2, b2r)


def kernel(x, edge_index, batch, W1, b1, W2, b2):
  src = edge_index[0]
  dst = edge_index[1]
  src3 = src.reshape(NW, NCHUNK, K)
  dst3 = dst.reshape(NW, NCHUNK, K)
  npad = NW * NCHB * K - E
  pad_idx = N + jnp.arange(npad, dtype=jnp.int32) % (NP - N)
  srcb = jnp.concatenate([src, pad_idx])
  dstb = jnp.concatenate([dst, pad_idx])
  src3b = srcb.reshape(NW, NCHB, K)
  dst3b = dstb.reshape(NW, NCHB, K)
  batch_p = jnp.concatenate([batch, jnp.full((NP - N,), G, jnp.int32)])
  batch3 = batch_p.reshape(NW, NBCH, K)
  batch3d = batch_p.reshape(NBLK, 1, RB)
  xp = jnp.concatenate([x, jnp.zeros((NP - N, D), jnp.float32)], axis=0)

  z640 = jnp.zeros((640,), jnp.float32)
  z256 = jnp.zeros((CNT_PAD,), jnp.float32)
  zsp = jnp.zeros((SP_SLICE,), jnp.float32)

  deg_pb, cnt_pb = _sc_hist(dst3, batch3, z640, z256)
  deg4 = deg_pb.reshape(NC, NBLK, 1, RB)

  u, dinv3, ac3, cntinv = _tc_prep(xp, W1, deg4, cnt_pb, batch3d)

  acc_pb = _sc_edge_agg(src3b, dst3b, u)
  sp_pb = _sc_sp(src3, dst3, batch_p, dinv3.reshape(NP),
                 cntinv.reshape(CNT_PAD), zsp)
  sp3 = sp_pb.reshape(NC, G, NP)

  return _tc_pool(acc_pb, u, dinv3, ac3, batch3d, sp3, cnt_pb,
                  b1.reshape(1, D), W2, b2.reshape(1, D))


# speedup vs baseline: 2.2549x; 1.0567x over previous
import jax
import jax.numpy as jnp
from jax import lax
from jax.experimental import pallas as pl
from jax.experimental.pallas import tpu as pltpu
from jax.experimental.pallas import tpu_sc as plsc

N = 10000
NP = 10240
E = 320000
D = 128
G = 100

NC = 2
NS = 16
NW = NC * NS
EPT = E // NW
K = 80
NCHUNK = EPT // K
BPT = NP // NW
NBCH = BPT // K
ROWS_PT = NP // NS
CNT_PAD = 256
SP_PAD = (G + 1) * NP
SP_SLICE = SP_PAD // NS
RB = NP // 8
NBLK = NP // RB


def _sc_mesh():
  return plsc.VectorSubcoreMesh(
      core_axis_name="c", subcore_axis_name="s", num_cores=NC, num_subcores=NS)


def _sc_params():
  return pltpu.CompilerParams(needs_layout_passes=False)


def _sc_hist_body(dst3, batch3, z640, z256, deg_out, cnt_out,
                  deg_sh, cnt_sh, dst_v, b_v, ones_v, sem):
  c = lax.axis_index("c")
  s = lax.axis_index("s")
  w = c * NS + s

  pltpu.sync_copy(z640, deg_sh.at[pl.ds(s * 640, 640)])

  @pl.when(s == 0)
  def _():
    pltpu.sync_copy(z256, cnt_sh)

  pltpu.sync_copy(dst3.at[w], dst_v)
  pltpu.sync_copy(batch3.at[w], b_v)
  for k in range(K // 16):
    ones_v[pl.ds(k * 16, 16)] = jnp.ones((16,), jnp.float32)

  plsc.subcore_barrier()

  @pl.loop(0, NCHUNK)
  def _(j):
    pltpu.async_copy(ones_v, deg_sh.at[dst_v.at[j]], sem, add=True)

  @pl.loop(0, NBCH)
  def _(j):
    pltpu.async_copy(ones_v, cnt_sh.at[b_v.at[j]], sem, add=True)

  @pl.loop(0, NCHUNK)
  def _(j):
    pltpu.make_async_copy(ones_v, deg_sh.at[dst_v.at[j]], sem).wait()

  @pl.loop(0, NBCH)
  def _(j):
    pltpu.make_async_copy(ones_v, cnt_sh.at[b_v.at[j]], sem).wait()

  plsc.subcore_barrier()

  pltpu.sync_copy(deg_sh.at[pl.ds(s * 640, 640)],
                  deg_out.at[c, pl.ds(s * 640, 640)])

  @pl.when(s == 0)
  def _():
    pltpu.sync_copy(cnt_sh, cnt_out.at[c])


def _sc_hist(dst3, batch3, z640, z256):
  return pl.kernel(
      _sc_hist_body,
      out_type=[
          jax.ShapeDtypeStruct((NC, NP), jnp.float32),
          jax.ShapeDtypeStruct((NC, CNT_PAD), jnp.float32),
      ],
      mesh=_sc_mesh(),
      scratch_types=[
          pltpu.VMEM_SHARED((NP,), jnp.float32),
          pltpu.VMEM_SHARED((CNT_PAD,), jnp.float32),
          pltpu.VMEM((NCHUNK, K), jnp.int32),
          pltpu.VMEM((NBCH, K), jnp.int32),
          pltpu.VMEM((K,), jnp.float32),
          pltpu.SemaphoreType.DMA,
      ],
      compiler_params=_sc_params(),
      name="sc_hist",
  )(dst3, batch3, z640, z256)


NCHB = 128
PH = 64


def _db_run(u, acc_sh, src_v, dst_v, rows_a, rows_b, sem_a, sem_b, count):
  pltpu.async_copy(u.at[src_v.at[0]], rows_a, sem_a)
  npairs = (count - 2) // 2

  @pl.loop(0, npairs)
  def _(i):
    j0 = 2 * i
    pltpu.async_copy(u.at[src_v.at[j0 + 1]], rows_b, sem_b)
    pltpu.make_async_copy(u.at[src_v.at[j0]], rows_a, sem_a).wait()
    pltpu.sync_copy(rows_a, acc_sh.at[dst_v.at[j0]], add=True)
    pltpu.async_copy(u.at[src_v.at[j0 + 2]], rows_a, sem_a)
    pltpu.make_async_copy(u.at[src_v.at[j0 + 1]], rows_b, sem_b).wait()
    pltpu.sync_copy(rows_b, acc_sh.at[dst_v.at[j0 + 1]], add=True)

  if count % 2 == 0:
    pltpu.async_copy(u.at[src_v.at[count - 1]], rows_b, sem_b)
    pltpu.make_async_copy(u.at[src_v.at[count - 2]], rows_a, sem_a).wait()
    pltpu.sync_copy(rows_a, acc_sh.at[dst_v.at[count - 2]], add=True)
    pltpu.make_async_copy(u.at[src_v.at[count - 1]], rows_b, sem_b).wait()
    pltpu.sync_copy(rows_b, acc_sh.at[dst_v.at[count - 1]], add=True)
  else:
    pltpu.async_copy(u.at[src_v.at[count - 2]], rows_b, sem_b)
    pltpu.make_async_copy(u.at[src_v.at[count - 3]], rows_a, sem_a).wait()
    pltpu.sync_copy(rows_a, acc_sh.at[dst_v.at[count - 3]], add=True)
    pltpu.async_copy(u.at[src_v.at[count - 1]], rows_a, sem_a)
    pltpu.make_async_copy(u.at[src_v.at[count - 2]], rows_b, sem_b).wait()
    pltpu.sync_copy(rows_b, acc_sh.at[dst_v.at[count - 2]], add=True)
    pltpu.make_async_copy(u.at[src_v.at[count - 1]], rows_a, sem_a).wait()
    pltpu.sync_copy(rows_a, acc_sh.at[dst_v.at[count - 1]], add=True)


def _sc_edge_agg_body(src3, dst3, u, acc_out,
                      acc_sh, src_v, dst_v, rows_a, rows_b, sem_a, sem_b):
  c = lax.axis_index("c")
  s = lax.axis_index("s")
  w = c * NS + s

  @pl.loop(0, K)
  def _(q):
    for l in range(D // 16):
      rows_a[q, pl.ds(l * 16, 16)] = jnp.zeros((16,), jnp.float32)

  for q in range(ROWS_PT // K):
    pltpu.sync_copy(rows_a, acc_sh.at[pl.ds(s * ROWS_PT + q * K, K)])

  pltpu.sync_copy(src3.at[w, pl.ds(0, PH)], src_v)
  pltpu.sync_copy(dst3.at[w, pl.ds(0, PH)], dst_v)

  plsc.subcore_barrier()

  _db_run(u, acc_sh, src_v, dst_v, rows_a, rows_b, sem_a, sem_b, PH)

  pltpu.sync_copy(src3.at[w, pl.ds(PH, PH)], src_v)
  pltpu.sync_copy(dst3.at[w, pl.ds(PH, PH)], dst_v)
  _db_run(u, acc_sh, src_v, dst_v, rows_a, rows_b, sem_a, sem_b, PH)

  plsc.subcore_barrier()

  for q in range(ROWS_PT // K):
    pltpu.sync_copy(acc_sh.at[pl.ds(s * ROWS_PT + q * K, K)], rows_a)
    pltpu.sync_copy(rows_a, acc_out.at[c, pl.ds(s * ROWS_PT + q * K, K)])


def _sc_edge_agg(src3, dst3, u):
  return pl.kernel(
      _sc_edge_agg_body,
      out_type=jax.ShapeDtypeStruct((NC, NP, D), jnp.float32),
      mesh=_sc_mesh(),
      scratch_types=[
          pltpu.VMEM_SHARED((NP, D), jnp.float32),
          pltpu.VMEM((PH, K), jnp.int32),
          pltpu.VMEM((PH, K), jnp.int32),
          pltpu.VMEM((K, D), jnp.float32),
          pltpu.VMEM((K, D), jnp.float32),
          pltpu.SemaphoreType.DMA,
          pltpu.SemaphoreType.DMA,
      ],
      compiler_params=_sc_params(),
      name="sc_edge_agg",
  )(src3, dst3, u)


PC = 32
NPC = NCHB // PC


def _sc_sp_body(src3, dst3, fb_h, c2_h, dinv_h, zsp, sp_out,
                sp_sh, src_v, dst_v, fb_v, c2_v, dinv_v,
                fbig0, vbig0, fbig1, vbig1, sem0, sem1):
  c = lax.axis_index("c")
  s = lax.axis_index("s")
  w = c * NS + s

  pltpu.sync_copy(zsp, sp_sh.at[pl.ds(s * SP_SLICE, SP_SLICE)])
  pltpu.sync_copy(fb_h, fb_v)
  pltpu.sync_copy(c2_h, c2_v)
  pltpu.sync_copy(dinv_h, dinv_v)

  plsc.subcore_barrier()

  bufs = [(fbig0, vbig0, sem0), (fbig1, vbig1, sem1)]
  for p in range(NPC):
    fbig, vbig, sem = bufs[p % 2]
    if p >= 2:
      @pl.loop(0, PC)
      def _(j):
        pltpu.make_async_copy(vbig.at[j], sp_sh.at[fbig.at[j]], sem).wait()

    pltpu.sync_copy(src3.at[w, pl.ds(p * PC, PC)], src_v)
    pltpu.sync_copy(dst3.at[w, pl.ds(p * PC, PC)], dst_v)

    @pl.loop(0, PC)
    def _(j):
      for k in range(K // 16):
        sv = src_v[j, pl.ds(k * 16, 16)]
        dv = dst_v[j, pl.ds(k * 16, 16)]
        val = plsc.load_gather(dinv_v, [sv]) * plsc.load_gather(c2_v, [dv])
        vbig[j, pl.ds(k * 16, 16)] = val
        fbig[j, pl.ds(k * 16, 16)] = plsc.load_gather(fb_v, [dv]) + sv

    @pl.loop(0, PC)
    def _(j):
      pltpu.async_copy(vbig.at[j], sp_sh.at[fbig.at[j]], sem, add=True)

  for p in (NPC - 2, NPC - 1):
    fbig, vbig, sem = bufs[p % 2]

    @pl.loop(0, PC)
    def _(j):
      pltpu.make_async_copy(vbig.at[j], sp_sh.at[fbig.at[j]], sem).wait()

  plsc.subcore_barrier()

  pltpu.sync_copy(sp_sh.at[pl.ds(s * SP_SLICE, SP_SLICE)],
                  sp_out.at[c, pl.ds(s * SP_SLICE, SP_SLICE)])


def _sc_sp(src3b, dst3b, fb_h, c2_h, dinv_h, zsp):
  return pl.kernel(
      _sc_sp_body,
      out_type=jax.ShapeDtypeStruct((NC, SP_PAD), jnp.float32),
      mesh=_sc_mesh(),
      scratch_types=[
          pltpu.VMEM_SHARED((SP_PAD,), jnp.float32),
          pltpu.VMEM((PC, K), jnp.int32),
          pltpu.VMEM((PC, K), jnp.int32),
          pltpu.VMEM((NP,), jnp.int32),
          pltpu.VMEM((NP,), jnp.float32),
          pltpu.VMEM((NP,), jnp.float32),
          pltpu.VMEM((PC, K), jnp.int32),
          pltpu.VMEM((PC, K), jnp.float32),
          pltpu.VMEM((PC, K), jnp.int32),
          pltpu.VMEM((PC, K), jnp.float32),
          pltpu.SemaphoreType.DMA,
          pltpu.SemaphoreType.DMA,
      ],
      compiler_params=_sc_params(),
      name="sc_sp",
  )(src3b, dst3b, fb_h, c2_h, dinv_h, zsp)


def _tc_prep_body(x_ref, w1_ref, deg_ref, cnt_ref, batch_ref,
                  u_ref, dinv_ref, ac_ref, fb_ref, c2_ref):
  deg = deg_ref[0, 0, 0, :] + deg_ref[1, 0, 0, :] + 1.0
  dinv = lax.rsqrt(deg)
  xw = jnp.dot(x_ref[...], w1_ref[...],
               preferred_element_type=jnp.float32)
  u_ref[...] = dinv[:, None] * xw
  dinv_ref[0, 0, :] = dinv

  cntsum = cnt_ref[0, :] + cnt_ref[1, :]
  cntinv = 1.0 / jnp.maximum(cntsum, 1.0)

  oh = (batch_ref[0, 0, :][:, None]
        == lax.broadcasted_iota(jnp.int32, (RB, CNT_PAD), 1))
  cpn = jnp.dot(oh.astype(jnp.float32), cntinv[:, None],
                preferred_element_type=jnp.float32)[:, 0]
  c2 = dinv * cpn
  c2_ref[0, 0, :] = c2
  ac_ref[0, 0, :] = dinv * c2
  fb_ref[0, 0, :] = batch_ref[0, 0, :] * NP


def _tc_prep(xp, W1, deg4, cnt, batch3d):
  return pl.pallas_call(
      _tc_prep_body,
      grid=(NBLK,),
      in_specs=[
          pl.BlockSpec((RB, D), lambda i: (i, 0)),
          pl.BlockSpec((D, D), lambda i: (0, 0)),
          pl.BlockSpec((NC, 1, 1, RB), lambda i: (0, i, 0, 0)),
          pl.BlockSpec((NC, CNT_PAD), lambda i: (0, 0)),
          pl.BlockSpec((1, 1, RB), lambda i: (i, 0, 0)),
      ],
      out_specs=[
          pl.BlockSpec((RB, D), lambda i: (i, 0)),
          pl.BlockSpec((1, 1, RB), lambda i: (i, 0, 0)),
          pl.BlockSpec((1, 1, RB), lambda i: (i, 0, 0)),
          pl.BlockSpec((1, 1, RB), lambda i: (i, 0, 0)),
          pl.BlockSpec((1, 1, RB), lambda i: (i, 0, 0)),
      ],
      out_shape=[
          jax.ShapeDtypeStruct((NP, D), jnp.float32),
          jax.ShapeDtypeStruct((NBLK, 1, RB), jnp.float32),
          jax.ShapeDtypeStruct((NBLK, 1, RB), jnp.float32),
          jax.ShapeDtypeStruct((NBLK, 1, RB), jnp.int32),
          jax.ShapeDtypeStruct((NBLK, 1, RB), jnp.float32),
      ],
      name="tc_prep",
  )(xp, W1, deg4, cnt, batch3d)


def _tc_pool_body(acc_ref, u_ref, dinv_ref, ac_ref, batch_ref, sp_ref,
                  cnt_ref, b1_ref, w2_ref, b2_ref, out_ref, accum):
  i = pl.program_id(0)
  accs = acc_ref[0] + acc_ref[1] + u_ref[...]
  h1 = jnp.maximum(dinv_ref[0, 0, :][:, None] * accs
                   + b1_ref[0, :][None, :], 0.0)
  iota_g = lax.broadcasted_iota(jnp.int32, (G, RB), 0)
  spb = sp_ref[0] + sp_ref[1] + jnp.where(
      batch_ref[0, 0, :][None, :] == iota_g,
      ac_ref[0, 0, :][None, :], 0.0)
  part = jnp.dot(spb, h1, preferred_element_type=jnp.float32)

  @pl.when(i == 0)
  def _():
    accum[...] = jnp.zeros_like(accum)

  accum[...] += part

  @pl.when(i == pl.num_programs(0) - 1)
  def _():
    cntsum = cnt_ref[0, :G] + cnt_ref[1, :G]
    mask = (cntsum > 0.0).astype(jnp.float32)[:, None]
    out_ref[...] = (jnp.dot(accum[...], w2_ref[...],
                            preferred_element_type=jnp.float32)
                    + mask * b2_ref[0, :][None, :])


def _tc_pool(acc_pb, u, dinv3, ac3, batch3d, sp3, cnt, b1r, W2, b2r):
  return pl.pallas_call(
      _tc_pool_body,
      grid=(NBLK,),
      in_specs=[
          pl.BlockSpec((NC, RB, D), lambda i: (0, i, 0)),
          pl.BlockSpec((RB, D), lambda i: (i, 0)),
          pl.BlockSpec((1, 1, RB), lambda i: (i, 0, 0)),
          pl.BlockSpec((1, 1, RB), lambda i: (i, 0, 0)),
          pl.BlockSpec((1, 1, RB), lambda i: (i, 0, 0)),
          pl.BlockSpec((NC, G, RB), lambda i: (0, 0, i)),
          pl.BlockSpec((NC, CNT_PAD), lambda i: (0, 0)),
          pl.BlockSpec((1, D), lambda i: (0, 0)),
          pl.BlockSpec((D, D), lambda i: (0, 0)),
          pl.BlockSpec((1, D), lambda i: (0, 0)),
      ],
      out_specs=pl.BlockSpec((G, D), lambda i: (0, 0)),
      out_shape=jax.ShapeDtypeStruct((G, D), jnp.float32),
      scratch_shapes=[pltpu.VMEM((G, D), jnp.float32)],
      name="tc_pool",
  )(acc_pb, u, dinv3, ac3, batch3d, sp3, cnt, b1r, W2, b2r)


def kernel(x, edge_index, batch, W1, b1, W2, b2):
  src = edge_index[0]
  dst = edge_index[1]
  src3 = src.reshape(NW, NCHUNK, K)
  dst3 = dst.reshape(NW, NCHUNK, K)
  npad = NW * NCHB * K - E
  pad_idx = N + jnp.arange(npad, dtype=jnp.int32) % (NP - N)
  srcb = jnp.concatenate([src, pad_idx])
  dstb = jnp.concatenate([dst, pad_idx])
  src3b = srcb.reshape(NW, NCHB, K)
  dst3b = dstb.reshape(NW, NCHB, K)
  batch_p = jnp.concatenate([batch, jnp.full((NP - N,), G, jnp.int32)])
  batch3 = batch_p.reshape(NW, NBCH, K)
  batch3d = batch_p.reshape(NBLK, 1, RB)
  xp = jnp.concatenate([x, jnp.zeros((NP - N, D), jnp.float32)], axis=0)

  z640 = jnp.zeros((640,), jnp.float32)
  z256 = jnp.zeros((CNT_PAD,), jnp.float32)
  zsp = jnp.zeros((SP_SLICE,), jnp.float32)

  deg_pb, cnt_pb = _sc_hist(dst3, batch3, z640, z256)
  deg4 = deg_pb.reshape(NC, NBLK, 1, RB)

  u, dinv3, ac3, fb3, c23 = _tc_prep(xp, W1, deg4, cnt_pb, batch3d)

  acc_pb = _sc_edge_agg(src3b, dst3b, u)
  sp_pb = _sc_sp(src3b, dst3b, fb3.reshape(NP), c23.reshape(NP),
                 dinv3.reshape(NP), zsp)
  sp3 = sp_pb[:, :G * NP].reshape(NC, G, NP)

  return _tc_pool(acc_pb, u, dinv3, ac3, batch3d, sp3, cnt_pb,
                  b1.reshape(1, D), W2, b2.reshape(1, D))
